# Initial kernel scaffold; baseline (speedup 1.0000x reference)
import jax
import jax.numpy as jnp
from jax.experimental import pallas as pl


def kernel(seq, tok_table, pos_table, gamma, beta):
    raise NotImplementedError("write your pallas kernel here")


# speedup vs baseline: 1.4553x; 1.4553x over previous
import functools

import jax
import jax.numpy as jnp
import numpy as np
from jax import lax
from jax.experimental import pallas as pl
from jax.experimental.pallas import tpu as pltpu
from jax.experimental.pallas import tpu_sc as plsc

DIM = 64
MAX_LEN = 200
NUM_CORES = 2
NUM_SUBCORES = 16
NUM_WORKERS = NUM_CORES * NUM_SUBCORES
LANES = 16
CHUNK = 256
IDX_ROW = 128
ROWS_PER_CHUNK = CHUNK // IDX_ROW
SCALE = 8.0
EPS = 1e-5


def _rsqrt(v):
    i = lax.bitcast_convert_type(v, jnp.int32)
    i = 0x5F3759DF - lax.shift_right_logical(i, 1)
    y = lax.bitcast_convert_type(i, jnp.float32)
    half = 0.5 * v
    for _ in range(3):
        y = y * (1.5 - half * y * y)
    return y


def _make_kernel(n_tokens):
    rows_total = n_tokens // IDX_ROW
    rows_per_worker = rows_total // NUM_WORKERS
    chunks_per_worker = rows_per_worker // ROWS_PER_CHUNK
    mesh = plsc.VectorSubcoreMesh(core_axis_name="c", subcore_axis_name="s")

    @functools.partial(
        pl.kernel,
        mesh=mesh,
        compiler_params=pltpu.CompilerParams(
            needs_layout_passes=False, use_tc_tiling_on_sc=False
        ),
        out_type=jax.ShapeDtypeStruct((n_tokens, DIM), jnp.float32),
        scratch_types=[
            pltpu.VMEM((ROWS_PER_CHUNK, IDX_ROW), jnp.int32),
            pltpu.VMEM((CHUNK, DIM), jnp.float32),
            pltpu.VMEM((CHUNK, DIM), jnp.float32),
            pltpu.VMEM((MAX_LEN, DIM), jnp.float32),
            pltpu.VMEM((DIM,), jnp.float32),
            pltpu.VMEM((DIM,), jnp.float32),
            pltpu.SemaphoreType.DMA,
        ],
    )
    def emb_kernel(seq_hbm, tok_hbm, pos_hbm, gamma_hbm, beta_hbm, out_hbm,
                   idx_v, rows_v, outb_v, pos_v, g_v, b_v, sem):
        wid = lax.axis_index("s") * NUM_CORES + lax.axis_index("c")

        pltpu.sync_copy(pos_hbm, pos_v)
        pltpu.sync_copy(gamma_hbm, g_v)
        pltpu.sync_copy(beta_hbm, b_v)

        g4 = [g_v[pl.ds(j * LANES, LANES)] for j in range(4)]
        b4 = [b_v[pl.ds(j * LANES, LANES)] for j in range(4)]

        row_base = wid * rows_per_worker

        def chunk_body(c, _):
            row0 = row_base + c * ROWS_PER_CHUNK
            pltpu.sync_copy(seq_hbm.at[pl.ds(row0, ROWS_PER_CHUNK)], idx_v)
            cps = [
                pltpu.async_copy(
                    tok_hbm.at[idx_v.at[r]],
                    rows_v.at[pl.ds(r * IDX_ROW, IDX_ROW)],
                    sem,
                )
                for r in range(ROWS_PER_CHUNK)
            ]
            for cp in cps:
                cp.wait()

            tok_base = row0 * IDX_ROW
            pbase = lax.rem(tok_base, MAX_LEN)

            def tok_body(tt, _):
                for u in range(4):
                    t = tt * 4 + u
                    p = lax.rem(pbase + t, MAX_LEN)
                    x = [rows_v[t, pl.ds(j * LANES, LANES)] * SCALE
                         + pos_v[p, pl.ds(j * LANES, LANES)]
                         for j in range(4)]
                    s = (x[0] + x[1]) + (x[2] + x[3])
                    sq = (x[0] * x[0] + x[1] * x[1]) + (x[2] * x[2] + x[3] * x[3])
                    tot = jnp.sum(s)
                    tot2 = jnp.sum(sq)
                    mean = tot * (1.0 / DIM)
                    var = jnp.maximum(tot2 * (1.0 / DIM) - mean * mean, 0.0)
                    inv = _rsqrt(var + EPS)
                    cc = mean * inv
                    for j in range(4):
                        y = (x[j] * inv - cc) * g4[j] + b4[j]
                        outb_v[t, pl.ds(j * LANES, LANES)] = y
                return 0

            lax.fori_loop(0, CHUNK // 4, tok_body, 0)
            pltpu.sync_copy(outb_v, out_hbm.at[pl.ds(tok_base, CHUNK)])
            return 0

        lax.fori_loop(0, chunks_per_worker, chunk_body, 0)

    return emb_kernel


@jax.jit
def kernel(seq, tok_table, pos_table, gamma, beta):
    b, s = seq.shape
    n = b * s
    seq2 = seq.reshape(n // IDX_ROW, IDX_ROW).astype(jnp.int32)
    out = _make_kernel(n)(seq2, tok_table, pos_table, gamma, beta)
    return out.reshape(b, s, DIM)


# speedup vs baseline: 3.5676x; 2.4515x over previous
import functools

import jax
import jax.numpy as jnp
from jax import lax
from jax.experimental import pallas as pl
from jax.experimental.pallas import tpu as pltpu
from jax.experimental.pallas import tpu_sc as plsc

DIM = 64
MAX_LEN = 200
NUM_CORES = 2
NUM_SUBCORES = 16
NUM_WORKERS = NUM_CORES * NUM_SUBCORES
LANES = 16
CHUNK = 256
IDX_ROW = 128
ROWS_PER_CHUNK = CHUNK // IDX_ROW
SCALE = 8.0
EPS = 1e-5


def _rsqrt(v):
    i = lax.bitcast_convert_type(v, jnp.int32)
    i = 0x5F3759DF - lax.shift_right_logical(i, 1)
    y = lax.bitcast_convert_type(i, jnp.float32)
    half = 0.5 * v
    for _ in range(3):
        y = y * (1.5 - half * y * y)
    return y


def _make_kernel(n_tokens):
    rows_total = n_tokens // IDX_ROW
    rows_per_worker = rows_total // NUM_WORKERS
    chunks_per_worker = rows_per_worker // ROWS_PER_CHUNK
    outer_iters = chunks_per_worker // 2
    mesh = plsc.VectorSubcoreMesh(core_axis_name="c", subcore_axis_name="s")

    @functools.partial(
        pl.kernel,
        mesh=mesh,
        compiler_params=pltpu.CompilerParams(
            needs_layout_passes=False, use_tc_tiling_on_sc=False
        ),
        out_type=jax.ShapeDtypeStruct((n_tokens, DIM), jnp.float32),
        scratch_types=[
            pltpu.VMEM((rows_per_worker, IDX_ROW), jnp.int32),
            pltpu.VMEM((2, CHUNK, DIM), jnp.float32),
            pltpu.VMEM((2, CHUNK, DIM), jnp.float32),
            pltpu.VMEM((MAX_LEN, DIM), jnp.float32),
            pltpu.VMEM((DIM,), jnp.float32),
            pltpu.VMEM((DIM,), jnp.float32),
            pltpu.SemaphoreType.DMA,
            pltpu.SemaphoreType.DMA,
            pltpu.SemaphoreType.DMA,
            pltpu.SemaphoreType.DMA,
        ],
    )
    def emb_kernel(seq_hbm, tok_hbm, pos_hbm, gamma_hbm, beta_hbm, out_hbm,
                   idx_all, rows, outb, pos_v, g_v, b_v,
                   sem_g0, sem_g1, sem_o0, sem_o1):
        wid = lax.axis_index("s") * NUM_CORES + lax.axis_index("c")
        row_base = wid * rows_per_worker

        pltpu.sync_copy(seq_hbm.at[pl.ds(row_base, rows_per_worker)], idx_all)
        pltpu.sync_copy(pos_hbm, pos_v)
        pltpu.sync_copy(gamma_hbm, g_v)
        pltpu.sync_copy(beta_hbm, b_v)

        g4 = [g_v[pl.ds(j * LANES, LANES)] for j in range(4)]
        b4 = [b_v[pl.ds(j * LANES, LANES)] for j in range(4)]

        sems_g = (sem_g0, sem_g1)
        sems_o = (sem_o0, sem_o1)

        def gather_copies(c, buf, sem):
            return [
                pltpu.make_async_copy(
                    tok_hbm.at[idx_all.at[c * ROWS_PER_CHUNK + r]],
                    rows.at[buf, pl.ds(r * IDX_ROW, IDX_ROW)],
                    sem,
                )
                for r in range(ROWS_PER_CHUNK)
            ]

        def fire_gather(c, buf, sem):
            for cp in gather_copies(c, buf, sem):
                cp.start()

        def wait_gather(c, buf, sem):
            for cp in gather_copies(c, buf, sem):
                cp.wait()

        def store_copy(tok0, buf, sem):
            return pltpu.make_async_copy(
                outb.at[buf], out_hbm.at[pl.ds(tok0, CHUNK)], sem
            )

        fire_gather(0, 0, sem_g0)

        def outer(g, _):
            for b in range(2):
                c = g * 2 + b
                nb = 1 - b
                if b == 0:
                    fire_gather(c + 1, nb, sems_g[nb])
                else:
                    @pl.when(g < outer_iters - 1)
                    def _():
                        fire_gather(c + 1, nb, sems_g[nb])
                wait_gather(c, b, sems_g[b])

                tok0 = (row_base + c * ROWS_PER_CHUNK) * IDX_ROW

                @pl.when(g > 0)
                def _():
                    store_copy(tok0, b, sems_o[b]).wait()

                pbase = lax.rem(tok0, MAX_LEN)

                @plsc.parallel_loop(0, CHUNK, 1, unroll=8)
                def tok_body(t):
                    p = lax.rem(pbase + t, MAX_LEN)
                    x = [rows[b, t, pl.ds(j * LANES, LANES)] * SCALE
                         + pos_v[p, pl.ds(j * LANES, LANES)]
                         for j in range(4)]
                    s = (x[0] + x[1]) + (x[2] + x[3])
                    sq = (x[0] * x[0] + x[1] * x[1]) + (x[2] * x[2] + x[3] * x[3])
                    mean = jnp.sum(s) * (1.0 / DIM)
                    var = jnp.sum(sq) * (1.0 / DIM) - mean * mean
                    inv = _rsqrt(var + EPS)
                    cc = mean * inv
                    for j in range(4):
                        outb[b, t, pl.ds(j * LANES, LANES)] = (
                            (x[j] * inv - cc) * g4[j] + b4[j]
                        )

                store_copy(tok0, b, sems_o[b]).start()
            return 0

        lax.fori_loop(0, outer_iters, outer, 0)

        for b in range(2):
            store_copy(b * CHUNK, b, sems_o[b]).wait()

    return emb_kernel


@jax.jit
def kernel(seq, tok_table, pos_table, gamma, beta):
    b, s = seq.shape
    n = b * s
    seq2 = seq.reshape(n // IDX_ROW, IDX_ROW).astype(jnp.int32)
    out = _make_kernel(n)(seq2, tok_table, pos_table, gamma, beta)
    return out.reshape(b, s, DIM)


# speedup vs baseline: 3.8184x; 1.0703x over previous
import functools

import jax
import jax.numpy as jnp
from jax import lax
from jax.experimental import pallas as pl
from jax.experimental.pallas import tpu as pltpu
from jax.experimental.pallas import tpu_sc as plsc

DIM = 64
NUM_CORES = 2
NUM_SUBCORES = 16
NUM_WORKERS = NUM_CORES * NUM_SUBCORES
LANES = 16
IDX_MAX = 128
SCALE = 8.0
EPS = 1e-5


def _rsqrt(v):
    i = lax.bitcast_convert_type(v, jnp.int32)
    i = 0x5F3759DF - lax.shift_right_logical(i, 1)
    y = lax.bitcast_convert_type(i, jnp.float32)
    half = 0.5 * v
    for _ in range(3):
        y = y * (1.5 - half * y * y)
    return y


def _make_kernel(batch, seqlen):
    rows_per_worker = batch // NUM_WORKERS
    outer_iters = rows_per_worker // 2
    segs = []
    off = 0
    while off < seqlen:
        n = min(IDX_MAX, seqlen - off)
        segs.append((off, n))
        off += n
    mesh = plsc.VectorSubcoreMesh(core_axis_name="c", subcore_axis_name="s")

    @functools.partial(
        pl.kernel,
        mesh=mesh,
        compiler_params=pltpu.CompilerParams(
            needs_layout_passes=False, use_tc_tiling_on_sc=False
        ),
        out_type=jax.ShapeDtypeStruct((batch, seqlen, DIM), jnp.float32),
        scratch_types=[
            pltpu.VMEM((rows_per_worker, seqlen), jnp.int32),
            pltpu.VMEM((2, seqlen, DIM), jnp.float32),
            pltpu.VMEM((2, seqlen, DIM), jnp.float32),
            pltpu.VMEM((seqlen, DIM), jnp.float32),
            pltpu.VMEM((DIM,), jnp.float32),
            pltpu.VMEM((DIM,), jnp.float32),
            pltpu.SemaphoreType.DMA,
            pltpu.SemaphoreType.DMA,
            pltpu.SemaphoreType.DMA,
            pltpu.SemaphoreType.DMA,
        ],
    )
    def emb_kernel(seq_hbm, tok_hbm, pos_hbm, gamma_hbm, beta_hbm, out_hbm,
                   idx_all, rows, outb, pos_v, g_v, b_v,
                   sem_g0, sem_g1, sem_o0, sem_o1):
        wid = lax.axis_index("s") * NUM_CORES + lax.axis_index("c")
        row_base = wid * rows_per_worker

        pltpu.sync_copy(seq_hbm.at[pl.ds(row_base, rows_per_worker)], idx_all)
        pltpu.sync_copy(pos_hbm, pos_v)
        pltpu.sync_copy(gamma_hbm, g_v)
        pltpu.sync_copy(beta_hbm, b_v)

        g4 = [g_v[pl.ds(j * LANES, LANES)] for j in range(4)]
        b4 = [b_v[pl.ds(j * LANES, LANES)] for j in range(4)]

        sems_g = (sem_g0, sem_g1)
        sems_o = (sem_o0, sem_o1)

        def gather_copies(r, buf, sem):
            return [
                pltpu.make_async_copy(
                    tok_hbm.at[idx_all.at[r, pl.ds(o, n)]],
                    rows.at[buf, pl.ds(o, n)],
                    sem,
                )
                for o, n in segs
            ]

        def fire_gather(r, buf, sem):
            for cp in gather_copies(r, buf, sem):
                cp.start()

        def wait_gather(r, buf, sem):
            for cp in gather_copies(r, buf, sem):
                cp.wait()

        def store_copy(bi, buf, sem):
            return pltpu.make_async_copy(outb.at[buf], out_hbm.at[bi], sem)

        fire_gather(0, 0, sem_g0)

        def outer(g, _):
            for b in range(2):
                r = g * 2 + b
                nb = 1 - b
                if b == 0:
                    fire_gather(r + 1, nb, sems_g[nb])
                else:
                    @pl.when(g < outer_iters - 1)
                    def _():
                        fire_gather(r + 1, nb, sems_g[nb])
                wait_gather(r, b, sems_g[b])

                bi = row_base + r

                @pl.when(g > 0)
                def _():
                    store_copy(bi, b, sems_o[b]).wait()

                @plsc.parallel_loop(0, seqlen, 1, unroll=8)
                def tok_body(t):
                    x = [rows[b, t, pl.ds(j * LANES, LANES)] * SCALE
                         + pos_v[t, pl.ds(j * LANES, LANES)]
                         for j in range(4)]
                    s = (x[0] + x[1]) + (x[2] + x[3])
                    sq = (x[0] * x[0] + x[1] * x[1]) + (x[2] * x[2] + x[3] * x[3])
                    mean = jnp.sum(s) * (1.0 / DIM)
                    var = jnp.sum(sq) * (1.0 / DIM) - mean * mean
                    inv = _rsqrt(var + EPS)
                    cc = mean * inv
                    for j in range(4):
                        outb[b, t, pl.ds(j * LANES, LANES)] = (
                            (x[j] * inv - cc) * g4[j] + b4[j]
                        )

                store_copy(bi, b, sems_o[b]).start()
            return 0

        lax.fori_loop(0, outer_iters, outer, 0)

        for b in range(2):
            store_copy(row_base + b, b, sems_o[b]).wait()

    return emb_kernel


@jax.jit
def kernel(seq, tok_table, pos_table, gamma, beta):
    b, s = seq.shape
    return _make_kernel(b, s)(
        seq.astype(jnp.int32), tok_table, pos_table, gamma, beta
    )
